# baseline (speedup 1.0000x reference)
import functools
import jax
import jax.numpy as jnp
from jax import lax
from jax.experimental import pallas as pl
from jax.experimental.pallas import tpu as pltpu
from jax.experimental.pallas import tpu_sc as plsc

_B, _S, _D = 4096, 50, 64
_NW = 32
_CHUNK = 128
_NCH = _S
_SCALE = 8.0
_LANES = 16
_NBUF = 3


def _body(tok_hbm, wp_hbm, out_hbm, idx_v, ibufs, obufs, gsems, ssems):
    c = lax.axis_index("c")
    s_ax = lax.axis_index("s")
    wid = s_ax * 2 + c
    pltpu.sync_copy(tok_hbm.at[:, wid], idx_v)

    def start_fetch(cg, b):
        pltpu.make_async_copy(
            wp_hbm.at[idx_v.at[cg]], ibufs[b], gsems[b]
        ).start()

    def scale(b):
        def row_body(r, _):
            for j in range(_D // _LANES):
                sl = pl.ds(j * _LANES, _LANES)
                obufs[b][r, sl] = ibufs[b][r, sl] * _SCALE
            return 0

        lax.fori_loop(0, _CHUNK, row_body, 0, unroll=8)

    for b in range(_NBUF):
        start_fetch(b, b)

    def step(cg, b):
        pltpu.make_async_copy(
            wp_hbm.at[idx_v.at[cg]], ibufs[b], gsems[b]
        ).wait()

        @pl.when(cg >= _NBUF)
        def _():
            pltpu.make_async_copy(
                obufs[b], out_hbm.at[pl.ds(0, _CHUNK), 0], ssems[b]
            ).wait()

        scale(b)

        @pl.when(cg + _NBUF < _NCH)
        def _():
            start_fetch(cg + _NBUF, b)

        pltpu.make_async_copy(
            obufs[b], out_hbm.at[pl.ds(_CHUNK * wid, _CHUNK), cg], ssems[b]
        ).start()

    def outer(g, _):
        for b in range(_NBUF):
            step(g * _NBUF + b, b)
        return 0

    full = _NCH // _NBUF
    lax.fori_loop(0, full, outer, 0)
    for b in range(_NCH - full * _NBUF):
        step(full * _NBUF + b, b)

    for b in range(_NBUF):
        pltpu.make_async_copy(
            obufs[b], out_hbm.at[pl.ds(0, _CHUNK), 0], ssems[b]
        ).wait()


_launch = functools.partial(
    pl.kernel,
    out_type=jax.ShapeDtypeStruct((_B, _S, _D), jnp.float32),
    mesh=plsc.VectorSubcoreMesh(core_axis_name="c", subcore_axis_name="s"),
    scratch_types=[
        pltpu.VMEM((_NCH, _CHUNK), jnp.int32),
        [pltpu.VMEM((_CHUNK, 2 * _D), jnp.float32) for _ in range(_NBUF)],
        [pltpu.VMEM((_CHUNK, _D), jnp.float32) for _ in range(_NBUF)],
        [pltpu.SemaphoreType.DMA for _ in range(_NBUF)],
        [pltpu.SemaphoreType.DMA for _ in range(_NBUF)],
    ],
    compiler_params=pltpu.CompilerParams(use_tc_tiling_on_sc=True),
)(_body)


def kernel(tokens, W):
    tok = tokens.T.reshape(_S, _NW, _CHUNK)
    wp = jnp.pad(W, ((0, 0), (0, _D)))
    return _launch(tok, wp)


# speedup vs baseline: 1.3055x; 1.3055x over previous
import functools
import jax
import jax.numpy as jnp
from jax import lax
from jax.experimental import pallas as pl
from jax.experimental.pallas import tpu as pltpu
from jax.experimental.pallas import tpu_sc as plsc

_B, _S, _D = 4096, 50, 64
_NW = 32
_CHUNK = 128
_NCH = _S
_SCALE = 8.0
_LANES = 16
_NBUF = 3


def _body(tok_hbm, w_hbm, out_hbm, idx_v, ibufs, obufs, gsems, ssems):
    c = lax.axis_index("c")
    s_ax = lax.axis_index("s")
    wid = s_ax * 2 + c
    pltpu.sync_copy(tok_hbm.at[:, wid], idx_v)

    def start_fetch(cg, b):
        def grp(g16, _):
            tv = idx_v[cg, pl.ds(g16 * _LANES, _LANES)]
            for l in range(_LANES):
                pltpu.make_async_copy(
                    w_hbm.at[tv[l]], ibufs[b].at[g16 * _LANES + l], gsems[b]
                ).start()
            return 0

        lax.fori_loop(0, _CHUNK // _LANES, grp, 0)

    def drain_fetch(b):
        pltpu.make_async_copy(
            w_hbm.at[pl.ds(0, _CHUNK)], ibufs[b], gsems[b]
        ).wait()

    def scale(b):
        def row_body(r, _):
            for j in range(_D // _LANES):
                sl = pl.ds(j * _LANES, _LANES)
                obufs[b][r, sl] = ibufs[b][r, sl] * _SCALE
            return 0

        lax.fori_loop(0, _CHUNK, row_body, 0, unroll=8)

    for b in range(_NBUF):
        start_fetch(b, b)

    def step(cg, b):
        drain_fetch(b)

        @pl.when(cg >= _NBUF)
        def _():
            pltpu.make_async_copy(
                obufs[b], out_hbm.at[pl.ds(0, _CHUNK), 0], ssems[b]
            ).wait()

        scale(b)

        @pl.when(cg + _NBUF < _NCH)
        def _():
            start_fetch(cg + _NBUF, b)

        pltpu.make_async_copy(
            obufs[b], out_hbm.at[pl.ds(_CHUNK * wid, _CHUNK), cg], ssems[b]
        ).start()

    def outer(g, _):
        for b in range(_NBUF):
            step(g * _NBUF + b, b)
        return 0

    full = _NCH // _NBUF
    lax.fori_loop(0, full, outer, 0)
    for b in range(_NCH - full * _NBUF):
        step(full * _NBUF + b, b)

    for b in range(_NBUF):
        pltpu.make_async_copy(
            obufs[b], out_hbm.at[pl.ds(0, _CHUNK), 0], ssems[b]
        ).wait()


_launch = functools.partial(
    pl.kernel,
    out_type=jax.ShapeDtypeStruct((_B, _S, _D), jnp.float32),
    mesh=plsc.VectorSubcoreMesh(core_axis_name="c", subcore_axis_name="s"),
    scratch_types=[
        pltpu.VMEM((_NCH, _CHUNK), jnp.int32),
        [pltpu.VMEM((_CHUNK, _D), jnp.float32) for _ in range(_NBUF)],
        [pltpu.VMEM((_CHUNK, _D), jnp.float32) for _ in range(_NBUF)],
        [pltpu.SemaphoreType.DMA for _ in range(_NBUF)],
        [pltpu.SemaphoreType.DMA for _ in range(_NBUF)],
    ],
    compiler_params=pltpu.CompilerParams(use_tc_tiling_on_sc=True),
)(_body)


def kernel(tokens, W):
    tok = tokens.T.reshape(_S, _NW, _CHUNK)
    return _launch(tok, W)
